# Initial kernel scaffold; baseline (speedup 1.0000x reference)
import jax
import jax.numpy as jnp
from jax.experimental import pallas as pl


def kernel(value_0, value_1, value_2, value_3, pos_0, pos_1, pos_2, pos_3, Wv, bv, Wo, bo, W_attn, b_attn, W_box, b_box):
    raise NotImplementedError("write your pallas kernel here")


# speedup vs baseline: 55.3787x; 55.3787x over previous
import functools

import jax
import jax.numpy as jnp
import numpy as np
from jax import lax
from jax.experimental import pallas as pl
from jax.experimental.pallas import tpu as pltpu
from jax.experimental.pallas import tpu_sc as plsc

D_MODEL = 256
NUM_HEADS = 8
NUM_LEVEL = 4
KERNEL = 2
NUM_POINT = KERNEL * KERNEL
REF_SIZE = 4
HEAD_DIM = D_MODEL // NUM_HEADS
SHAPES = [(64, 64), (32, 32), (16, 16), (8, 8)]
B = 2
L = sum(h * w for h, w in SHAPES)
LVL_START = [0, 4096, 5120, 5376]

TOK_BLK = 320
N_BLK = L // TOK_BLK
N_TOK = B * L
NTAP = NUM_HEADS * NUM_LEVEL * NUM_POINT * 4
NROWS = B * L * NUM_HEADS


def _np_kernel_indices():
    idx = np.linspace(-0.5, 0.5, KERNEL)
    i, j = np.meshgrid(idx, idx, indexing='ij')
    return np.stack([j, i], axis=-1).reshape(-1, 2) / KERNEL


def _np_ref_windows():
    eps = 1e-6
    refs = []
    for (H, W) in SHAPES:
        y = (np.arange(1, H + 1, dtype=np.float64) - 0.5) / (H + eps)
        x = (np.arange(1, W + 1, dtype=np.float64) - 0.5) / (W + eps)
        yy, xx = np.meshgrid(y.astype(np.float32), x.astype(np.float32),
                             indexing='ij')
        center = np.stack([xx, yy], axis=-1).reshape(-1, 2)
        wh = np.broadcast_to(
            np.array([REF_SIZE / W, REF_SIZE / H], np.float32), center.shape)
        refs.append(np.concatenate([center, wh], axis=-1).astype(np.float32))
    return np.concatenate(refs, axis=0)


def _np_col_consts():
    kid = _np_kernel_indices()
    kx = np.zeros((128,), np.float32)
    ky = np.zeros((128,), np.float32)
    wl = np.zeros((128,), np.float32)
    hl = np.zeros((128,), np.float32)
    ls = np.zeros((128,), np.float32)
    hc = np.zeros((128,), np.float32)
    for h in range(NUM_HEADS):
        for lvl in range(NUM_LEVEL):
            for p in range(NUM_POINT):
                c = (h * 4 + lvl) * 4 + p
                kx[c] = kid[p, 0]
                ky[c] = kid[p, 1]
                wl[c] = SHAPES[lvl][1]
                hl[c] = SHAPES[lvl][0]
                ls[c] = LVL_START[lvl]
                hc[c] = h
    return np.stack([kx, ky, wl, hl, ls, hc], axis=0)


def _np_sel():
    s = np.zeros((128, 512), np.float32)
    for k in range(4):
        for j in range(128):
            s[(j // 4) * 4 + k, k * 128 + j] = 1.0
    return s


def _np_gmat():
    g = np.zeros((128, 128), np.float32)
    for i in range(128):
        for j in range(128):
            if i // 16 == j // 16:
                g[i, j] = 1.0
    return g


_REF_WIN = _np_ref_windows()
_CC = _np_col_consts()
_SEL = _np_sel()
_G = _np_gmat()


def _prep_body(flat_ref, posf_ref, ref_ref, Wv_ref, bv_ref, Wa_ref, ba_ref,
               Wb_ref, bb_ref, sel_ref, g_ref, cc_ref,
               table_ref, attn_ref, idx_ref, coef_ref):
    b = pl.program_id(0)
    flat = flat_ref[0]
    q = flat + posf_ref[0]

    dot = functools.partial(lax.dot_general,
                            dimension_numbers=(((1,), (1,)), ((), ())),
                            preferred_element_type=jnp.float32)
    dotn = functools.partial(lax.dot_general,
                             dimension_numbers=(((1,), (0,)), ((), ())),
                             preferred_element_type=jnp.float32)

    table_ref[0] = dot(flat, Wv_ref[...]) + bv_ref[0]

    logits = dot(q, Wa_ref[...]) + ba_ref[0]
    m = jnp.max(logits, axis=1, keepdims=True)
    e = jnp.exp(logits - m)
    attn = e / dotn(e, g_ref[...])
    attn_ref[0] = attn

    offs = dot(q, Wb_ref[...]) + bb_ref[0]
    offs4 = dotn(offs, sel_ref[...])
    ox = offs4[:, 0:128]
    oy = offs4[:, 128:256]
    ow = offs4[:, 256:384]
    oh = offs4[:, 384:512]

    cx = ref_ref[:, 0:1]
    cy = ref_ref[:, 1:2]
    wr = ref_ref[:, 2:3]
    hr = ref_ref[:, 3:4]

    kx = cc_ref[0:1, :]
    ky = cc_ref[1:2, :]
    wl = cc_ref[2:3, :]
    hl = cc_ref[3:4, :]
    ls = cc_ref[4:5, :]
    hc = cc_ref[5:6, :]

    bx = cx + ox * (wr * 0.125)
    by = cy + oy * (hr * 0.125)
    rw = jnp.maximum(wr + ow * (wr * 0.125), 0.0)
    rh = jnp.maximum(hr + oh * (hr * 0.125), 0.0)
    gx = bx + kx * rw
    gy = by + ky * rh
    px = gx * wl - 0.5
    py = gy * hl - 0.5
    x0f = jnp.floor(px)
    y0f = jnp.floor(py)
    fx = px - x0f
    fy = py - y0f
    x0 = x0f.astype(jnp.int32)
    y0 = y0f.astype(jnp.int32)
    wli = wl.astype(jnp.int32)
    hli = hl.astype(jnp.int32)
    lsi = ls.astype(jnp.int32)
    hci = hc.astype(jnp.int32)
    rowbase = lsi + b * L

    corners = ((0, 0, (1.0 - fx) * (1.0 - fy)),
               (1, 0, fx * (1.0 - fy)),
               (0, 1, (1.0 - fx) * fy),
               (1, 1, fx * fy))
    for ci, (dx, dy, cw) in enumerate(corners):
        xx = x0 + dx
        yy = y0 + dy
        valid = ((xx >= 0) & (xx < wli) & (yy >= 0) & (yy < hli))
        xc = jnp.clip(xx, 0, wli - 1)
        yc = jnp.clip(yy, 0, hli - 1)
        row = (rowbase + yc * wli + xc) * NUM_HEADS + hci
        coef = cw * valid.astype(jnp.float32) * attn
        idx_ref[0, :, ci, :] = row
        coef_ref[0, :, ci, :] = coef


def _prep(flat, posf, Wv, bv, Wa, ba, Wb, bb):
    T = TOK_BLK
    refw = jnp.asarray(_REF_WIN)
    sel = jnp.asarray(_SEL)
    g = jnp.asarray(_G)
    cc = jnp.asarray(_CC)
    grid = (B, N_BLK)
    out_shapes = (
        jax.ShapeDtypeStruct((B, L, D_MODEL), jnp.float32),
        jax.ShapeDtypeStruct((B, L, 128), jnp.float32),
        jax.ShapeDtypeStruct((B, L, 4, 128), jnp.int32),
        jax.ShapeDtypeStruct((B, L, 4, 128), jnp.float32),
    )
    full2 = lambda s: pl.BlockSpec(s, lambda b, i: (0, 0))
    return pl.pallas_call(
        _prep_body,
        grid=grid,
        in_specs=[
            pl.BlockSpec((1, T, D_MODEL), lambda b, i: (b, i, 0)),
            pl.BlockSpec((1, T, D_MODEL), lambda b, i: (b, i, 0)),
            pl.BlockSpec((T, 4), lambda b, i: (i, 0)),
            full2((D_MODEL, D_MODEL)),
            full2((1, D_MODEL)),
            full2((128, D_MODEL)),
            full2((1, 128)),
            full2((128, D_MODEL)),
            full2((1, 128)),
            full2((128, 512)),
            full2((128, 128)),
            full2((6, 128)),
        ],
        out_specs=[
            pl.BlockSpec((1, T, D_MODEL), lambda b, i: (b, i, 0)),
            pl.BlockSpec((1, T, 128), lambda b, i: (b, i, 0)),
            pl.BlockSpec((1, T, 4, 128), lambda b, i: (b, i, 0, 0)),
            pl.BlockSpec((1, T, 4, 128), lambda b, i: (b, i, 0, 0)),
        ],
        out_shape=out_shapes,
    )(flat, posf, refw, Wv, bv.reshape(1, -1), Wa, ba.reshape(1, -1),
      Wb, bb.reshape(1, -1), sel, g, cc)


_NC = 2
_NS = 16
_NW = _NC * _NS
_TPW = N_TOK // _NW


def _sc_body(idx_hbm, coef_hbm, table_hbm, out_hbm,
             idx_v, coef_v, rows_v, out_v, gsem):
    wid = lax.axis_index("s") * _NC + lax.axis_index("c")
    base = wid * _TPW

    def body(t, carry):
        tok = base + t
        pltpu.sync_copy(idx_hbm.at[tok], idx_v)
        pltpu.sync_copy(coef_hbm.at[tok], coef_v)
        copies = [
            pltpu.async_copy(table_hbm.at[idx_v.at[j]], rows_v.at[j], gsem)
            for j in range(4)
        ]
        for cp in copies:
            cp.wait()
        for h in range(NUM_HEADS):
            acc0 = jnp.zeros((16,), jnp.float32)
            acc1 = jnp.zeros((16,), jnp.float32)
            for c in range(4):
                cvec = coef_v[pl.ds(c * 128 + h * 16, 16)]
                for gidx in range(16):
                    o = h * 16 + gidx
                    s = cvec[gidx]
                    acc0 = acc0 + s * rows_v[c, o, 0:16]
                    acc1 = acc1 + s * rows_v[c, o, 16:32]
            out_v[pl.ds(h * 32, 16)] = acc0
            out_v[pl.ds(h * 32 + 16, 16)] = acc1
        pltpu.sync_copy(out_v, out_hbm.at[tok])
        return carry

    lax.fori_loop(0, _TPW, body, 0)


def _sc_sample(idx, coef, table):
    mesh = plsc.VectorSubcoreMesh(core_axis_name="c", subcore_axis_name="s")
    f = pl.kernel(
        _sc_body,
        mesh=mesh,
        out_type=jax.ShapeDtypeStruct((N_TOK, D_MODEL), jnp.float32),
        scratch_types=[
            pltpu.VMEM((4, 128), jnp.int32),
            pltpu.VMEM((NTAP,), jnp.float32),
            pltpu.VMEM((4, 128, HEAD_DIM), jnp.float32),
            pltpu.VMEM((D_MODEL,), jnp.float32),
            pltpu.SemaphoreType.DMA,
        ],
        compiler_params=pltpu.CompilerParams(use_tc_tiling_on_sc=False),
    )
    return f(idx, coef, table)


def _oproj_body(x_ref, Wo_ref, bo_ref, o_ref):
    o_ref[0] = lax.dot_general(
        x_ref[0], Wo_ref[...], dimension_numbers=(((1,), (1,)), ((), ())),
        preferred_element_type=jnp.float32) + bo_ref[0]


def _oproj(x, Wo, bo):
    return pl.pallas_call(
        _oproj_body,
        grid=(B, N_BLK),
        in_specs=[
            pl.BlockSpec((1, TOK_BLK, D_MODEL), lambda b, i: (b, i, 0)),
            pl.BlockSpec((D_MODEL, D_MODEL), lambda b, i: (0, 0)),
            pl.BlockSpec((1, D_MODEL), lambda b, i: (0, 0)),
        ],
        out_specs=pl.BlockSpec((1, TOK_BLK, D_MODEL), lambda b, i: (b, i, 0)),
        out_shape=jax.ShapeDtypeStruct((B, L, D_MODEL), jnp.float32),
    )(x, Wo, bo.reshape(1, -1))


def kernel(value_0, value_1, value_2, value_3, pos_0, pos_1, pos_2, pos_3,
           Wv, bv, Wo, bo, W_attn, b_attn, W_box, b_box):
    values = [value_0, value_1, value_2, value_3]
    poss = [pos_0, pos_1, pos_2, pos_3]
    flat = jnp.concatenate(
        [v.reshape(B, D_MODEL, -1).transpose(0, 2, 1) for v in values], axis=1)
    posf = jnp.concatenate(
        [p.reshape(B, D_MODEL, -1).transpose(0, 2, 1) for p in poss], axis=1)

    table, attn, idx, coef = _prep(flat, posf, Wv, bv, W_attn, b_attn,
                                   W_box, b_box)

    sampled = _sc_sample(idx.reshape(N_TOK, 4, 128),
                         coef.reshape(N_TOK, NTAP),
                         table.reshape(NROWS, HEAD_DIM))

    out = _oproj(sampled.reshape(B, L, D_MODEL), Wo, bo)
    return out, attn.reshape(B, L, NUM_HEADS, NUM_LEVEL, KERNEL, KERNEL)
